# baseline (speedup 1.0000x reference)
import functools

import jax
import jax.numpy as jnp
from jax import lax
from jax.experimental import pallas as pl
from jax.experimental.pallas import tpu as pltpu
from jax.experimental.pallas import tpu_sc as plsc

_NC, _NS, _L = 2, 16, 16
_NW = _NC * _NS
_ROWS = 2 * 512 * 40
_COLS = 512
_ROWS_PER_W = _ROWS // _NW
_CROWS = 16
_NCHUNK = _ROWS_PER_W // _CROWS
_CGROUPS = _COLS // 16
_CWORDS = _CROWS * _COLS // 4

_LP = (1.4709377958029698e-06, 0.99984772525462, -0.4973734077959428,
       0.31574786947421507, -0.19035508404628199, 0.0826916958575542,
       -0.01741417587554822)


def _vgather(v, idx):
    return lax.gather(
        v, idx[:, None],
        lax.GatherDimensionNumbers(
            offset_dims=(), collapsed_slice_dims=(0,), start_index_map=(0,)),
        slice_sizes=(1,),
        mode=lax.GatherScatterMode.PROMISE_IN_BOUNDS)


def _log1p_poly(u):
    acc = jnp.full((16,), _LP[6], jnp.float32)
    for c in (_LP[5], _LP[4], _LP[3], _LP[2], _LP[1], _LP[0]):
        acc = acc * u + c
    return acc


def _sc_body(x_hbm, w_hbm, pm_hbm, out_hbm, xb, wb, pmb, accb):
    wid = lax.axis_index("s") * _NC + lax.axis_index("c")
    row_base = wid * _ROWS_PER_W
    word_base = row_base * (_COLS // 4)
    lanes = lax.iota(jnp.int32, 16)
    lane_div4 = lax.shift_right_logical(lanes, 2)
    byte_shift = (lanes & 3) * 8

    def chunk_body(k, carry):
        r0 = pl.multiple_of(row_base + k * _CROWS, _CROWS)
        pltpu.sync_copy(x_hbm.at[pl.ds(r0, _CROWS), :], xb)
        pltpu.sync_copy(w_hbm.at[pl.ds(r0, _CROWS), :], wb)
        pltpu.sync_copy(pm_hbm.at[pl.ds(r0, _CROWS), :], pmb)

        def row_body(r, carry2):
            def col_body(c, carry3):
                accn3, accd3 = carry3
                wstart = jnp.minimum(4 * c, (_COLS // 4) - 16)
                rel = 4 * c - wstart
                pmw16 = pmb[r, pl.ds(wstart, 16)]
                gidx = rel + lane_div4
                pmw = _vgather(pmw16, gidx)
                pmbyte = lax.shift_right_logical(pmw, byte_shift) & 0xFF
                pbit = pmbyte & 0xF
                mbit = lax.shift_right_logical(pmbyte, 4)
                xv = xb[r, pl.ds(c * 16, 16)]
                wv = wb[r, pl.ds(c * 16, 16)]
                tsel = pbit != 0
                wm = jnp.where(mbit != 0, wv, 0.0)
                z = jnp.where(tsel, -xv, xv)
                u = jnp.exp(-jnp.abs(xv))
                d = 1.0 + u
                rr = 1.0 / d
                pt = jnp.where(z >= 0.0, rr, 1.0 - rr)
                bce = jnp.maximum(z, 0.0) + _log1p_poly(u)
                aw = jnp.where(tsel, 0.25, 0.75)
                accn3 = accn3 + (aw * wm) * (pt * pt) * bce
                accd3 = accd3 + wm
                return accn3, accd3

            return lax.fori_loop(0, _CGROUPS, col_body, carry2)

        return lax.fori_loop(0, _CROWS, row_body, carry)

    zero = jnp.zeros((16,), jnp.float32)
    accn, accd = lax.fori_loop(0, _NCHUNK, chunk_body, (zero, zero))
    accb[0, :] = accn
    accb[1, :] = accd
    pltpu.sync_copy(accb, out_hbm.at[pl.ds(wid * 2, 2)])


_sc_call = functools.partial(
    pl.kernel,
    mesh=plsc.VectorSubcoreMesh(core_axis_name="c", subcore_axis_name="s"),
    out_type=jax.ShapeDtypeStruct((_NW * 2, 16), jnp.float32),
    scratch_types=[
        pltpu.VMEM((_CROWS, _COLS), jnp.float32),
        pltpu.VMEM((_CROWS, _COLS), jnp.float32),
        pltpu.VMEM((_CROWS, _COLS // 4), jnp.int32),
        pltpu.VMEM((2, 16), jnp.float32),
    ],
)(_sc_body)


def _as_native_2d(a):
    b, d1, d2, d3 = a.shape
    return a.transpose(0, 1, 3, 2).reshape(b * d1 * d3, d2)


def _pack_words(pos4d, m4d):
    b, d1, d2, d3 = pos4d.shape
    p5 = pos4d.reshape(b, d1, d2 // 4, 4, d3)
    m5 = m4d.reshape(b, d1, d2 // 4, 4, d3)
    consts = jnp.array([1, 1 << 8, 1 << 16, 1 << 24], jnp.int32)
    cc = consts[None, None, None, :, None]
    w = jnp.sum(p5 * cc + m5 * (cc << 4), axis=3)
    return w.transpose(0, 1, 3, 2).reshape(b * d1 * d3, d2 // 4)


def kernel(pred_occ_logit, general_cls_loss_mask_float, pos_mask, general_cls_loss_mask):
    b, _, d1, d2, d3 = pred_occ_logit.shape
    x2 = _as_native_2d(pred_occ_logit.reshape(b, d1, d2, d3))
    w2 = _as_native_2d(general_cls_loss_mask_float)
    pm = _pack_words(pos_mask, general_cls_loss_mask)
    parts = _sc_call(x2, w2, pm)
    num = jnp.sum(parts[0::2])
    den = jnp.sum(parts[1::2])
    return num / jnp.maximum(den, 1.0)


# speedup vs baseline: 4.3596x; 4.3596x over previous
import functools

import jax
import jax.numpy as jnp
from jax import lax
from jax.experimental import pallas as pl
from jax.experimental.pallas import tpu as pltpu
from jax.experimental.pallas import tpu_sc as plsc

_LANES = 512
_ROWS = 2 * 512 * 40
_SC_D1 = 448
_TC_ROWS = (512 + _SC_D1) * 40
_SC_ROWS = _ROWS - _TC_ROWS
_BLOCK_ROWS = 1920

_NC, _NS = 2, 16
_NW = _NC * _NS
_ROWS_PER_W = _SC_ROWS // _NW
_CROWS = 16
_NCHUNK = _ROWS_PER_W // _CROWS
_CGROUPS = _LANES // 16

_LP = (1.4709377958029698e-06, 0.99984772525462, -0.4973734077959428,
       0.31574786947421507, -0.19035508404628199, 0.0826916958575542,
       -0.01741417587554822)


def _focal_block_kernel(x_ref, w_ref, pm_ref, out_ref, accn_ref, accd_ref):
    i = pl.program_id(0)

    @pl.when(i == 0)
    def _init():
        accn_ref[...] = jnp.zeros_like(accn_ref)
        accd_ref[...] = jnp.zeros_like(accd_ref)

    def body(j, carry):
        an, ad = carry
        x = x_ref[pl.ds(j * 8, 8), :]
        pm = pm_ref[pl.ds(j * 8, 8), :].astype(jnp.int32)
        tb = (pm & 1) > 0
        wm = w_ref[pl.ds(j * 8, 8), :] * (pm >> 4).astype(jnp.float32)

        z = jnp.where(tb, -x, x)
        u = jnp.exp(-jnp.abs(x))
        d = 1.0 + u
        r = 1.0 / d
        pt = jnp.where(z >= 0.0, r, 1.0 - r)
        bce = jnp.maximum(z, 0.0) + jnp.log(d)
        alpha_w = jnp.where(tb, 0.25, 0.75)
        contrib = (alpha_w * wm) * (pt * pt) * bce
        return an + contrib, ad + wm

    zero = jnp.zeros((8, _LANES), jnp.float32)
    an, ad = jax.lax.fori_loop(0, _BLOCK_ROWS // 8, body, (zero, zero),
                               unroll=8)
    accn_ref[...] += an
    accd_ref[...] += ad

    @pl.when(i == pl.num_programs(0) - 1)
    def _finish():
        out_ref[0, 0] = jnp.sum(accn_ref[...])
        out_ref[0, 1] = jnp.sum(accd_ref[...])


def _tc_call(x2, w2, pm2):
    grid = _TC_ROWS // _BLOCK_ROWS
    return pl.pallas_call(
        _focal_block_kernel,
        grid=(grid,),
        in_specs=[
            pl.BlockSpec((_BLOCK_ROWS, _LANES), lambda i: (i, 0)),
            pl.BlockSpec((_BLOCK_ROWS, _LANES), lambda i: (i, 0)),
            pl.BlockSpec((_BLOCK_ROWS, _LANES), lambda i: (i, 0)),
        ],
        out_specs=pl.BlockSpec((1, 2), lambda i: (0, 0),
                               memory_space=pltpu.SMEM),
        out_shape=jax.ShapeDtypeStruct((1, 2), jnp.float32),
        scratch_shapes=[
            pltpu.VMEM((8, _LANES), jnp.float32),
            pltpu.VMEM((8, _LANES), jnp.float32),
        ],
    )(x2, w2, pm2)


def _log1p_poly(u):
    acc = jnp.full((16,), _LP[6], jnp.float32)
    for c in (_LP[5], _LP[4], _LP[3], _LP[2], _LP[1], _LP[0]):
        acc = acc * u + c
    return acc


def _vgather(v, idx):
    return lax.gather(
        v, idx[:, None],
        lax.GatherDimensionNumbers(
            offset_dims=(), collapsed_slice_dims=(0,), start_index_map=(0,)),
        slice_sizes=(1,),
        mode=lax.GatherScatterMode.PROMISE_IN_BOUNDS)


def _sc_body(x_hbm, w_hbm, pm_hbm, out_hbm, xb, wb, pmb, accb):
    wid = lax.axis_index("s") * _NC + lax.axis_index("c")
    row_base = _TC_ROWS + wid * _ROWS_PER_W
    wrow_base = wid * _ROWS_PER_W
    lanes = lax.iota(jnp.int32, 16)
    lane_div4 = lax.shift_right_logical(lanes, 2)
    byte_shift = (lanes & 3) * 8

    def chunk_body(k, carry):
        r0 = pl.multiple_of(row_base + k * _CROWS, _CROWS)
        wr0 = pl.multiple_of(wrow_base + k * _CROWS, _CROWS)
        pltpu.sync_copy(x_hbm.at[pl.ds(r0, _CROWS), :], xb)
        pltpu.sync_copy(w_hbm.at[pl.ds(r0, _CROWS), :], wb)
        pltpu.sync_copy(pm_hbm.at[pl.ds(wr0, _CROWS), :], pmb)

        def row_body(r, carry2):
            def col_body(c, carry3):
                accn3, accd3 = carry3
                wstart = jnp.minimum(4 * c, (_LANES // 4) - 16)
                rel = 4 * c - wstart
                pmw16 = pmb[r, pl.ds(wstart, 16)]
                gidx = rel + lane_div4
                pmw = _vgather(pmw16, gidx)
                pmbyte = lax.shift_right_logical(pmw, byte_shift) & 0xFF
                pbit = pmbyte & 0xF
                mbit = lax.shift_right_logical(pmbyte, 4)
                xv = xb[r, pl.ds(c * 16, 16)]
                wv = wb[r, pl.ds(c * 16, 16)]
                tsel = pbit != 0
                wm = jnp.where(mbit != 0, wv, 0.0)
                z = jnp.where(tsel, -xv, xv)
                u = jnp.exp(-jnp.abs(xv))
                d = 1.0 + u
                rr = 1.0 / d
                pt = jnp.where(z >= 0.0, rr, 1.0 - rr)
                bce = jnp.maximum(z, 0.0) + _log1p_poly(u)
                aw = jnp.where(tsel, 0.25, 0.75)
                accn3 = accn3 + (aw * wm) * (pt * pt) * bce
                accd3 = accd3 + wm
                return accn3, accd3

            return lax.fori_loop(0, _CGROUPS, col_body, carry2)

        return lax.fori_loop(0, _CROWS, row_body, carry)

    zero = jnp.zeros((16,), jnp.float32)
    accn, accd = lax.fori_loop(0, _NCHUNK, chunk_body, (zero, zero))
    accb[0, :] = accn
    accb[1, :] = accd
    pltpu.sync_copy(accb, out_hbm.at[pl.ds(wid * 2, 2)])


_sc_call = functools.partial(
    pl.kernel,
    mesh=plsc.VectorSubcoreMesh(core_axis_name="c", subcore_axis_name="s"),
    out_type=jax.ShapeDtypeStruct((_NW * 2, 16), jnp.float32),
    scratch_types=[
        pltpu.VMEM((_CROWS, _LANES), jnp.float32),
        pltpu.VMEM((_CROWS, _LANES), jnp.float32),
        pltpu.VMEM((_CROWS, _LANES // 4), jnp.int32),
        pltpu.VMEM((2, 16), jnp.float32),
    ],
)(_sc_body)


def _as_native_2d(a):
    b, d1, d2, d3 = a.shape
    return a.transpose(0, 1, 3, 2).reshape(b * d1 * d3, d2)


def _pack_words_2d(pos4d, m4d):
    b, d1, d2, d3 = pos4d.shape
    p5 = pos4d.reshape(b, d1, d2 // 4, 4, d3)
    m5 = m4d.reshape(b, d1, d2 // 4, 4, d3)
    consts = jnp.array([1, 1 << 8, 1 << 16, 1 << 24], jnp.int32)
    cc = consts[None, None, None, :, None]
    w = jnp.sum(p5 * cc + m5 * (cc << 4), axis=3)
    return w.transpose(0, 1, 3, 2).reshape(b * d1 * d3, d2 // 4)


def kernel(pred_occ_logit, general_cls_loss_mask_float, pos_mask, general_cls_loss_mask):
    b, _, d1, d2, d3 = pred_occ_logit.shape
    x2 = _as_native_2d(pred_occ_logit.reshape(b, d1, d2, d3))
    w2 = _as_native_2d(general_cls_loss_mask_float)
    pm2 = _as_native_2d(pos_mask.astype(jnp.int8)
                        | (general_cls_loss_mask.astype(jnp.int8) << 4))
    pmw = _pack_words_2d(pos_mask[1:, _SC_D1:],
                         general_cls_loss_mask[1:, _SC_D1:])

    sc_parts = _sc_call(x2, w2, pmw)
    tc_part = _tc_call(x2, w2, pm2)

    num = tc_part[0, 0] + jnp.sum(sc_parts[0::2])
    den = tc_part[0, 1] + jnp.sum(sc_parts[1::2])
    return num / jnp.maximum(den, 1.0)
